# Initial kernel scaffold; baseline (speedup 1.0000x reference)
import jax
import jax.numpy as jnp
from jax.experimental import pallas as pl


def kernel(u_X, c_S, W, b, keys, values, semantic_embeddings):
    raise NotImplementedError("write your pallas kernel here")


# speedup vs baseline: 5.8708x; 5.8708x over previous
import functools

import jax
import jax.numpy as jnp
from jax.experimental import pallas as pl

_B = 4096
_KB = 1000
_KB_PAD = 1024
_TOPK = 5
_TEMP_INV = 1.0 / 0.07
_BLK = 256
_NBLK = _B // _BLK
_NEG = -1e30


def _kern(x_ref, c_ref, wt_ref, b_ref, k_ref, v_ref, sem_ref,
          e_ref, idx_ref, cost_ref):
    i = pl.program_id(0)
    x = x_ref[...]
    q = jnp.dot(x, wt_ref[...], preferred_element_type=jnp.float32) + b_ref[...]
    q = jnp.maximum(q, 0.0)
    qn = q / jnp.maximum(
        jnp.sqrt(jnp.sum(q * q, axis=-1, keepdims=True)), 1e-12)

    k = k_ref[...]
    kn = k / jnp.maximum(
        jnp.sqrt(jnp.sum(k * k, axis=-1, keepdims=True)), 1e-12)

    s = jnp.dot(qn, kn.T, preferred_element_type=jnp.float32) * _TEMP_INV
    col = jax.lax.broadcasted_iota(jnp.int32, s.shape, 1)
    s = jnp.where(col < _KB, s, _NEG)

    walpha = jnp.zeros_like(s)
    denom = jnp.zeros((s.shape[0], 1), jnp.float32)
    idx_cols = []
    m0 = None
    s_cur = s
    for j in range(_TOPK):
        m = jnp.max(s_cur, axis=-1, keepdims=True)
        idx = jnp.min(jnp.where(s_cur == m, col, jnp.int32(1 << 30)),
                      axis=-1, keepdims=True)
        onehot = (col == idx)
        if j == 0:
            m0 = m
        w = jnp.exp(m - m0)
        walpha = walpha + jnp.where(onehot, w, 0.0)
        denom = denom + w
        idx_cols.append(idx)
        s_cur = jnp.where(onehot, _NEG, s_cur)
    walpha = walpha / denom

    e_ref[...] = jnp.dot(walpha, v_ref[...], preferred_element_type=jnp.float32)
    idx_ref[...] = jnp.concatenate(
        idx_cols + [jnp.zeros((s.shape[0], 8 - _TOPK), jnp.int32)], axis=1)

    cs = c_ref[...]
    csn = cs / jnp.maximum(
        jnp.sqrt(jnp.sum(cs * cs, axis=-1, keepdims=True)), 1e-12)
    sem = sem_ref[...]
    semn = sem / jnp.maximum(
        jnp.sqrt(jnp.sum(sem * sem, axis=-1, keepdims=True)), 1e-12)
    cmat = jnp.dot(csn, semn.T, preferred_element_type=jnp.float32)
    part = jnp.sum(walpha * (1.0 - cmat)) * (1.0 / _B)

    @pl.when(i == 0)
    def _():
        cost_ref[...] = jnp.zeros_like(cost_ref)

    cost_ref[...] += part


@functools.partial(jax.jit, static_argnames=())
def kernel(u_X, c_S, W, b, keys, values, semantic_embeddings):
    x = jnp.concatenate([u_X, c_S], axis=-1)
    wt = W.T
    b2 = b.reshape(1, -1)
    pad = _KB_PAD - _KB
    keys_p = jnp.pad(keys, ((0, pad), (0, 0)))
    values_p = jnp.pad(values, ((0, pad), (0, 0)))
    sem_p = jnp.pad(semantic_embeddings, ((0, pad), (0, 0)))

    d = x.shape[1]
    dk = keys.shape[1]
    dv = values.shape[1]

    e_out, idx_out, cost_out = pl.pallas_call(
        _kern,
        grid=(_NBLK,),
        in_specs=[
            pl.BlockSpec((_BLK, d), lambda i: (i, 0)),
            pl.BlockSpec((_BLK, c_S.shape[1]), lambda i: (i, 0)),
            pl.BlockSpec((d, W.shape[0]), lambda i: (0, 0)),
            pl.BlockSpec((1, W.shape[0]), lambda i: (0, 0)),
            pl.BlockSpec((_KB_PAD, dk), lambda i: (0, 0)),
            pl.BlockSpec((_KB_PAD, dv), lambda i: (0, 0)),
            pl.BlockSpec((_KB_PAD, c_S.shape[1]), lambda i: (0, 0)),
        ],
        out_specs=[
            pl.BlockSpec((_BLK, dv), lambda i: (i, 0)),
            pl.BlockSpec((_BLK, 8), lambda i: (i, 0)),
            pl.BlockSpec((1, 1), lambda i: (0, 0)),
        ],
        out_shape=[
            jax.ShapeDtypeStruct((_B, dv), jnp.float32),
            jax.ShapeDtypeStruct((_B, 8), jnp.int32),
            jax.ShapeDtypeStruct((1, 1), jnp.float32),
        ],
    )(x, c_S, wt, b2, keys_p, values_p, sem_p)

    return (e_out, idx_out[:, :_TOPK], cost_out[0, 0])


# speedup vs baseline: 6.7461x; 1.1491x over previous
import functools

import jax
import jax.numpy as jnp
from jax.experimental import pallas as pl

_B = 4096
_KB = 1000
_KB_PAD = 1024
_TOPK = 5
_TEMP_INV = 1.0 / 0.07
_BLK = 512
_NBLK = _B // _BLK
_NEG = -1e30


def _prep_kern(keys_ref, sem_ref, kn_ref, semn_ref):
    k = keys_ref[...]
    kn_ref[...] = k * (_TEMP_INV / jnp.maximum(
        jnp.sqrt(jnp.sum(k * k, axis=-1, keepdims=True)), 1e-12))
    s = sem_ref[...]
    semn_ref[...] = s / jnp.maximum(
        jnp.sqrt(jnp.sum(s * s, axis=-1, keepdims=True)), 1e-12)


def _main_kern(x_ref, c_ref, wt_ref, b_ref, kn_ref, vs_ref,
               e_ref, idx_ref, cost_ref):
    i = pl.program_id(0)
    x = x_ref[...]
    q = jnp.dot(x, wt_ref[...], preferred_element_type=jnp.float32) + b_ref[...]
    q = jnp.maximum(q, 0.0)
    qn = q / jnp.maximum(
        jnp.sqrt(jnp.sum(q * q, axis=-1, keepdims=True)), 1e-12)

    s = jnp.dot(qn, kn_ref[...].T, preferred_element_type=jnp.float32)
    col = jax.lax.broadcasted_iota(jnp.int32, s.shape, 1)
    s = jnp.where(col < _KB, s, _NEG)

    walpha = jnp.zeros_like(s)
    denom = jnp.zeros((s.shape[0], 1), jnp.float32)
    idx_cols = []
    m0 = None
    s_cur = s
    for j in range(_TOPK):
        m = jnp.max(s_cur, axis=-1, keepdims=True)
        idx = jnp.min(jnp.where(s_cur == m, col, jnp.int32(1 << 30)),
                      axis=-1, keepdims=True)
        onehot = (col == idx)
        if j == 0:
            m0 = m
        w = jnp.exp(m - m0)
        walpha = walpha + jnp.where(onehot, w, 0.0)
        denom = denom + w
        idx_cols.append(idx)
        s_cur = jnp.where(onehot, _NEG, s_cur)
    walpha = walpha / denom

    eg = jnp.dot(walpha, vs_ref[...], preferred_element_type=jnp.float32)
    dv = e_ref.shape[1]
    e_ref[...] = eg[:, :dv]
    g = eg[:, dv:]
    idx_ref[...] = jnp.concatenate(
        idx_cols + [jnp.zeros((s.shape[0], 8 - _TOPK), jnp.int32)], axis=1)

    cs = c_ref[...]
    csn = cs / jnp.maximum(
        jnp.sqrt(jnp.sum(cs * cs, axis=-1, keepdims=True)), 1e-12)
    part = jnp.sum(1.0 - jnp.sum(csn * g, axis=-1)) * (1.0 / _B)

    @pl.when(i == 0)
    def _():
        cost_ref[...] = jnp.zeros_like(cost_ref)

    cost_ref[...] += part


@jax.jit
def kernel(u_X, c_S, W, b, keys, values, semantic_embeddings):
    x = jnp.concatenate([u_X, c_S], axis=-1)
    wt = W.T
    b2 = b.reshape(1, -1)
    pad = _KB_PAD - _KB
    keys_p = jnp.pad(keys, ((0, pad), (0, 0)))
    sem_p = jnp.pad(semantic_embeddings, ((0, pad), (0, 0)))

    d = x.shape[1]
    dk = keys.shape[1]
    dv = values.shape[1]
    dsem = semantic_embeddings.shape[1]

    kn, semn = pl.pallas_call(
        _prep_kern,
        out_shape=[
            jax.ShapeDtypeStruct((_KB_PAD, dk), jnp.float32),
            jax.ShapeDtypeStruct((_KB_PAD, dsem), jnp.float32),
        ],
    )(keys_p, sem_p)

    values_p = jnp.pad(values, ((0, pad), (0, 0)))
    vs = jnp.concatenate([values_p, semn], axis=1)

    e_out, idx_out, cost_out = pl.pallas_call(
        _main_kern,
        grid=(_NBLK,),
        in_specs=[
            pl.BlockSpec((_BLK, d), lambda i: (i, 0)),
            pl.BlockSpec((_BLK, c_S.shape[1]), lambda i: (i, 0)),
            pl.BlockSpec((d, W.shape[0]), lambda i: (0, 0)),
            pl.BlockSpec((1, W.shape[0]), lambda i: (0, 0)),
            pl.BlockSpec((_KB_PAD, dk), lambda i: (0, 0)),
            pl.BlockSpec((_KB_PAD, dv + dsem), lambda i: (0, 0)),
        ],
        out_specs=[
            pl.BlockSpec((_BLK, dv), lambda i: (i, 0)),
            pl.BlockSpec((_BLK, 8), lambda i: (i, 0)),
            pl.BlockSpec((1, 1), lambda i: (0, 0)),
        ],
        out_shape=[
            jax.ShapeDtypeStruct((_B, dv), jnp.float32),
            jax.ShapeDtypeStruct((_B, 8), jnp.int32),
            jax.ShapeDtypeStruct((1, 1), jnp.float32),
        ],
    )(x, c_S, wt, b2, kn, vs)

    return (e_out, idx_out[:, :_TOPK], cost_out[0, 0])
